# baseline (speedup 1.0000x reference)
import functools

import jax
import jax.numpy as jnp
from jax import lax
from jax.experimental import pallas as pl
from jax.experimental.pallas import tpu as pltpu
from jax.experimental.pallas import tpu_sc as plsc

GRID_ROWS, GRID_COLS = 128, 128
SDF_SIZE = GRID_ROWS * GRID_COLS
COLS = 3 * SDF_SIZE + 6
PARAM_BASE = 3 * SDF_SIZE
BATCH = 1024
NC, NS, L = 2, 16, 16
NW = NC * NS
RPW = BATCH // NW

_mesh = plsc.VectorSubcoreMesh(core_axis_name="c", subcore_axis_name="s")


@functools.partial(
    pl.kernel,
    out_type=jax.ShapeDtypeStruct((BATCH,), jnp.float32),
    mesh=_mesh,
    scratch_types=[
        pltpu.VMEM((6, RPW), jnp.int32),
        pltpu.VMEM((6, RPW), jnp.float32),
        pltpu.VMEM((RPW,), jnp.int32),
        pltpu.VMEM((RPW,), jnp.float32),
        pltpu.VMEM((RPW,), jnp.float32),
        pltpu.SemaphoreType.DMA,
    ],
)
def _sdf_lookup(xf, out, pidx, pval, vidx, vval, omask, sem):
    wid = lax.axis_index("s") * NC + lax.axis_index("c")
    base = wid * RPW
    lanes = lax.iota(jnp.int32, L)

    for h in range(RPW // L):
        sl = pl.ds(h * L, L)
        rowoff = (base + h * L + lanes) * COLS + PARAM_BASE
        for j in range(6):
            pidx[j, sl] = rowoff + j

    copies = [
        pltpu.async_copy(xf.at[pidx.at[j]], pval.at[j], sem) for j in range(6)
    ]
    for c in copies:
        c.wait()

    for h in range(RPW // L):
        sl = pl.ds(h * L, L)
        res0, res1 = pval[0, sl], pval[1, sl]
        org0, org1 = pval[2, sl], pval[3, sl]
        ip0, ip1 = pval[4, sl], pval[5, sl]

        i0 = (ip0 / res0 + org0).astype(jnp.int32)
        i1 = (ip1 / res1 + org1).astype(jnp.int32)
        flat = i0 * GRID_COLS + i1
        oob = (i0 < 0) | (i0 >= GRID_COLS) | (i1 < 0) | (i1 >= GRID_ROWS)
        safe = jnp.clip(flat, 0, SDF_SIZE - 1)
        vidx[sl] = (base + h * L + lanes) * COLS + safe
        omask[sl] = jnp.where(oob, jnp.float32(1.0), jnp.float32(0.0))

    pltpu.async_copy(xf.at[vidx], vval, sem).wait()

    for h in range(RPW // L):
        sl = pl.ds(h * L, L)
        vval[sl] = jnp.where(omask[sl] > 0.5, jnp.float32(-0.1), vval[sl])

    pltpu.sync_copy(vval, out.at[pl.ds(base, RPW)])


def kernel(x):
    xf = x.reshape(-1)
    return _sdf_lookup(xf)[:, None]


# speedup vs baseline: 11.6506x; 11.6506x over previous
import functools

import jax
import jax.numpy as jnp
from jax import lax
from jax.experimental import pallas as pl
from jax.experimental.pallas import tpu as pltpu
from jax.experimental.pallas import tpu_sc as plsc

GRID_ROWS, GRID_COLS = 128, 128
SDF_SIZE = GRID_ROWS * GRID_COLS
COLS = 3 * SDF_SIZE + 6
PARAM_BASE = 3 * SDF_SIZE
BATCH = 1024
NC, NS, L = 2, 16, 16
NW = NC * NS
RPW = BATCH // NW

_mesh = plsc.VectorSubcoreMesh(core_axis_name="c", subcore_axis_name="s")


@functools.partial(
    pl.kernel,
    out_type=jax.ShapeDtypeStruct((BATCH,), jnp.float32),
    mesh=_mesh,
    scratch_types=[
        pltpu.VMEM((6, RPW), jnp.float32),
        pltpu.VMEM((RPW, 8, 128), jnp.float32),
        pltpu.VMEM((RPW,), jnp.float32),
        pltpu.SemaphoreType.DMA,
    ],
)
def _sdf_lookup(x2d, pflat, out, pvals, vbuf, obuf, sem):
    wid = lax.axis_index("s") * NC + lax.axis_index("c")
    base = wid * RPW
    lanes = lax.iota(jnp.int32, L)

    copies = [
        pltpu.async_copy(pflat.at[pl.ds(j * BATCH + base, RPW)], pvals.at[j], sem)
        for j in range(6)
    ]
    for c in copies:
        c.wait()

    ctiles, cmods, oobs = [], [], []
    for h in range(RPW // L):
        sl = pl.ds(h * L, L)
        res0, res1 = pvals[0, sl], pvals[1, sl]
        org0, org1 = pvals[2, sl], pvals[3, sl]
        ip0, ip1 = pvals[4, sl], pvals[5, sl]

        i0 = (ip0 / res0 + org0).astype(jnp.int32)
        i1 = (ip1 / res1 + org1).astype(jnp.int32)
        flat = i0 * GRID_COLS + i1
        oob = (i0 < 0) | (i0 >= GRID_COLS) | (i1 < 0) | (i1 >= GRID_ROWS)
        safe = jnp.clip(flat, 0, SDF_SIZE - 1)
        ctiles.append(safe & ~127)
        cmods.append(safe & 127)
        oobs.append(oob)

    vcopies = []
    for i in range(RPW):
        h, l = divmod(i, L)
        vcopies.append(
            pltpu.async_copy(
                x2d.at[pl.ds(pl.multiple_of(base + (i & ~7), 8), 8),
                       pl.ds(pl.multiple_of(ctiles[h][l], 128), 128)],
                vbuf.at[i],
                sem,
            )
        )
    for c in vcopies:
        c.wait()

    for h in range(RPW // L):
        sl = pl.ds(h * L, L)
        vals = jnp.full((L,), 0.0, jnp.float32)
        for l in range(L):
            i = h * L + l
            c = cmods[h][l]
            v16 = vbuf[i, i & 7, pl.ds(c & ~15, L)]
            g = v16.at[jnp.full((L,), c & 15, jnp.int32)].get(
                mode="promise_in_bounds"
            )
            vals = jnp.where(lanes == l, g, vals)
        obuf[sl] = jnp.where(oobs[h], jnp.float32(-0.1), vals)

    pltpu.sync_copy(obuf, out.at[pl.ds(base, RPW)])


def kernel(x):
    pflat = x[:, PARAM_BASE:PARAM_BASE + 6].T.reshape(-1)
    return _sdf_lookup(x, pflat)[:, None]


# speedup vs baseline: 88.1528x; 7.5664x over previous
import functools

import jax
import jax.numpy as jnp
from jax import lax
from jax.experimental import pallas as pl
from jax.experimental.pallas import tpu as pltpu
from jax.experimental.pallas import tpu_sc as plsc

GRID_ROWS, GRID_COLS = 128, 128
SDF_SIZE = GRID_ROWS * GRID_COLS
COLS = 3 * SDF_SIZE + 6
PARAM_BASE = 3 * SDF_SIZE
BATCH = 1024
NC, NS, L = 2, 16, 16
NW = NC * NS
RPW = BATCH // NW

_mesh = plsc.VectorSubcoreMesh(core_axis_name="c", subcore_axis_name="s")


@functools.partial(
    pl.kernel,
    out_type=jax.ShapeDtypeStruct((BATCH,), jnp.float32),
    mesh=_mesh,
    scratch_types=[
        pltpu.VMEM((6, RPW), jnp.float32),
        pltpu.VMEM((RPW, 8, 128), jnp.float32),
        pltpu.VMEM((RPW,), jnp.float32),
        pltpu.SemaphoreType.DMA,
    ],
)
def _sdf_lookup(xT, pflat, out, pvals, vbuf, obuf, sem):
    wid = lax.axis_index("s") * NC + lax.axis_index("c")
    base = wid * RPW
    rband = pl.multiple_of(base & ~127, 128)
    boff = base & 127
    lanes = lax.iota(jnp.int32, L)

    copies = [
        pltpu.async_copy(pflat.at[pl.ds(j * BATCH + base, RPW)], pvals.at[j], sem)
        for j in range(6)
    ]
    for c in copies:
        c.wait()

    safes, oobs = [], []
    for h in range(RPW // L):
        sl = pl.ds(h * L, L)
        res0, res1 = pvals[0, sl], pvals[1, sl]
        org0, org1 = pvals[2, sl], pvals[3, sl]
        ip0, ip1 = pvals[4, sl], pvals[5, sl]

        i0 = (ip0 / res0 + org0).astype(jnp.int32)
        i1 = (ip1 / res1 + org1).astype(jnp.int32)
        flat = i0 * GRID_COLS + i1
        oob = (i0 < 0) | (i0 >= GRID_COLS) | (i1 < 0) | (i1 >= GRID_ROWS)
        safes.append(jnp.clip(flat, 0, SDF_SIZE - 1))
        oobs.append(oob)

    vcopies = []
    for i in range(RPW):
        h, l = divmod(i, L)
        ct = pl.multiple_of(safes[h][l] & ~7, 8)
        vcopies.append(
            pltpu.async_copy(
                xT.at[pl.ds(ct, 8), pl.ds(rband, 128)], vbuf.at[i], sem
            )
        )
    for c in vcopies:
        c.wait()

    for h in range(RPW // L):
        sl = pl.ds(h * L, L)
        off = pl.multiple_of(boff + h * L, L)
        vals = jnp.full((L,), 0.0, jnp.float32)
        for l in range(L):
            i = h * L + l
            v16 = vbuf[i, safes[h][l] & 7, pl.ds(off, L)]
            vals = jnp.where(lanes == l, v16, vals)
        obuf[sl] = jnp.where(oobs[h], jnp.float32(-0.1), vals)

    pltpu.sync_copy(obuf, out.at[pl.ds(base, RPW)])


def kernel(x):
    pflat = x[:, PARAM_BASE:PARAM_BASE + 6].T.reshape(-1)
    return _sdf_lookup(x.T, pflat)[:, None]


# speedup vs baseline: 89.3891x; 1.0140x over previous
import functools

import jax
import jax.numpy as jnp
from jax import lax
from jax.experimental import pallas as pl
from jax.experimental.pallas import tpu as pltpu
from jax.experimental.pallas import tpu_sc as plsc

GRID_ROWS, GRID_COLS = 128, 128
SDF_SIZE = GRID_ROWS * GRID_COLS
COLS = 3 * SDF_SIZE + 6
PARAM_BASE = 3 * SDF_SIZE
BATCH = 1024
NC, NS, L = 2, 16, 16
NW = NC * NS
RPW = BATCH // NW

_mesh = plsc.VectorSubcoreMesh(core_axis_name="c", subcore_axis_name="s")


@functools.partial(
    pl.kernel,
    out_type=jax.ShapeDtypeStruct((BATCH,), jnp.float32),
    mesh=_mesh,
    scratch_types=[
        pltpu.VMEM((6 * RPW,), jnp.float32),
        pltpu.VMEM((RPW, 8, 128), jnp.float32),
        pltpu.VMEM((RPW + L,), jnp.int32),
        pltpu.VMEM((RPW + L,), jnp.int32),
        pltpu.VMEM((RPW,), jnp.float32),
        pltpu.SemaphoreType.DMA,
    ],
)
def _sdf_lookup(xT, pflat, out, pvals, vbuf, cbuf, sbuf, obuf, sem):
    wid = lax.axis_index("s") * NC + lax.axis_index("c")
    base = wid * RPW
    rband = pl.multiple_of(base & ~127, 128)
    boff = base & 127
    lanes = lax.iota(jnp.int32, L)

    pltpu.async_copy(
        pflat.at[pl.ds(wid * (6 * RPW), 6 * RPW)], pvals, sem
    ).wait()

    oobs = []
    for h in range(RPW // L):
        sl = pl.ds(h * L, L)

        def p(j, _h=h):
            return pvals[pl.ds(j * RPW + _h * L, L)]

        i0 = (p(4) / p(0) + p(2)).astype(jnp.int32)
        i1 = (p(5) / p(1) + p(3)).astype(jnp.int32)
        flat = i0 * GRID_COLS + i1
        oob = (i0 < 0) | (i0 >= GRID_COLS) | (i1 < 0) | (i1 >= GRID_ROWS)
        safe = jnp.clip(flat, 0, SDF_SIZE - 1)
        cbuf[sl] = safe & ~7
        sbuf[sl] = safe & 7
        oobs.append(oob)

    def fire(i, _):
        ct = pl.multiple_of(cbuf[pl.ds(i, L)][0], 8)
        pltpu.async_copy(
            xT.at[pl.ds(ct, 8), pl.ds(rband, 128)], vbuf.at[i], sem
        )
        return _

    lax.fori_loop(0, RPW, fire, 0, unroll=False)

    def drain(i, _):
        pltpu.make_async_copy(
            xT.at[pl.ds(0, 8), pl.ds(rband, 128)], vbuf.at[i], sem
        ).wait()
        return _

    lax.fori_loop(0, RPW, drain, 0, unroll=False)

    for h in range(RPW // L):
        sl = pl.ds(h * L, L)
        off = pl.multiple_of(boff + h * L, L)

        def extract(l, acc, _h=h, _off=off):
            i = _h * L + l
            v16 = vbuf[i, sbuf[pl.ds(i, L)][0], pl.ds(_off, L)]
            return jnp.where(lanes == l, v16, acc)

        vals = lax.fori_loop(0, L, extract, jnp.full((L,), 0.0, jnp.float32),
                             unroll=False)
        obuf[sl] = jnp.where(oobs[h], jnp.float32(-0.1), vals)

    pltpu.sync_copy(obuf, out.at[pl.ds(base, RPW)])


def kernel(x):
    pflat = (
        x[:, PARAM_BASE:PARAM_BASE + 6]
        .reshape(NW, RPW, 6)
        .transpose(0, 2, 1)
        .reshape(-1)
    )
    return _sdf_lookup(x.T, pflat)[:, None]
